# baseline (speedup 1.0000x reference)
import functools

import jax
import jax.numpy as jnp
from jax import lax
from jax.experimental import pallas as pl
from jax.experimental.pallas import tpu as pltpu
from jax.experimental.pallas import tpu_sc as plsc

N = 50000
F = 32
NC = 2
NS = 16
NW = NC * NS
SUB = 128
KSUB = 2
CH = SUB * KSUB
NPAD = 50048
ZR = NPAD // NS

_MESH = dict(core_axis_name="c", subcore_axis_name="s")
_SC_PARAMS = pltpu.CompilerParams(use_tc_tiling_on_sc=False)


def _pad_len(m):
    blk = 2 * NW * CH
    return ((m + blk - 1) // blk) * blk


def _pad_gather_idx(idx, mpad):
    pad = mpad - idx.shape[0]
    fill = lax.iota(jnp.int32, pad) % 128
    return jnp.concatenate([idx.astype(jnp.int32), fill]).reshape(mpad // SUB, SUB)


def _pad_scatter_idx(idx, mpad):
    pad = mpad - idx.shape[0]
    fill = N + (lax.iota(jnp.int32, pad) % 16)
    return jnp.concatenate([idx.astype(jnp.int32), fill]).reshape(mpad // SUB, SUB)


def _prop_scratch(width):
    return [
        pltpu.VMEM((KSUB, SUB), jnp.int32),
        pltpu.VMEM((KSUB, SUB), jnp.int32),
        pltpu.VMEM((KSUB, SUB), jnp.int32),
        pltpu.VMEM((KSUB, SUB), jnp.int32),
        pltpu.VMEM((KSUB, SUB, width), jnp.float32),
        pltpu.VMEM((KSUB, SUB, width), jnp.float32),
        pltpu.VMEM_SHARED((NPAD, width), jnp.float32),
        pltpu.SemaphoreType.DMA,
        pltpu.SemaphoreType.DMA,
    ]


def _edge_loop(load_idx, gather_tab, acc, bufs, nchunks):
    gi0, gi1, si0, si1, rows0, rows1, sem_g, ss = bufs

    def fire(j, gib, sib, rows):
        load_idx(j, gib, sib)
        gets = [
            pltpu.async_copy(gather_tab.at[gib.at[k]], rows.at[k], sem_g)
            for k in range(KSUB)
        ]
        for cp in gets:
            cp.wait()
        for k in range(KSUB):
            pltpu.async_copy(rows.at[k], acc.at[sib.at[k]], ss, add=True)

    def drain(sib, rows):
        for k in range(KSUB):
            pltpu.make_async_copy(rows.at[k], acc.at[sib.at[k]], ss).wait()

    def pair(t, carry):
        @pl.when(t >= 1)
        def _():
            drain(si0, rows0)

        fire(2 * t, gi0, si0, rows0)

        @pl.when(t >= 1)
        def _():
            drain(si1, rows1)

        fire(2 * t + 1, gi1, si1, rows1)
        return carry

    lax.fori_loop(0, nchunks // 2, pair, 0)
    drain(si0, rows0)
    drain(si1, rows1)


@functools.lru_cache(maxsize=None)
def _make_prop(mpad, width):
    rows_per_w = mpad // SUB // NW
    nchunks = rows_per_w // KSUB

    @functools.partial(
        pl.kernel,
        out_type=jax.ShapeDtypeStruct((NC, NPAD, width), jnp.float32),
        mesh=plsc.VectorSubcoreMesh(**_MESH),
        scratch_types=_prop_scratch(width),
        compiler_params=_SC_PARAMS,
    )
    def prop(table, gidx, sidx, zrows, out,
             gi0, gi1, si0, si1, rows0, rows1, acc, sem_g, ss):
        c = lax.axis_index("c")
        s = lax.axis_index("s")
        w = c * NS + s
        pltpu.sync_copy(zrows, acc.at[pl.ds(s * ZR, ZR)])
        plsc.subcore_barrier()
        base = w * rows_per_w

        def load_idx(j, gib, sib):
            r0 = base + j * KSUB
            pltpu.sync_copy(gidx.at[pl.ds(r0, KSUB)], gib)
            pltpu.sync_copy(sidx.at[pl.ds(r0, KSUB)], sib)

        bufs = (gi0, gi1, si0, si1, rows0, rows1, sem_g, ss)
        _edge_loop(load_idx, table, acc, bufs, nchunks)
        plsc.subcore_barrier()
        pltpu.sync_copy(acc.at[pl.ds(s * ZR, ZR)], out.at[c, pl.ds(s * ZR, ZR)])

    return prop


@functools.lru_cache(maxsize=None)
def _make_dual(mpad):
    rows_per_t = mpad // SUB // NS
    nchunks = rows_per_t // KSUB

    @functools.partial(
        pl.kernel,
        out_type=jax.ShapeDtypeStruct((NC, NPAD, F), jnp.float32),
        mesh=plsc.VectorSubcoreMesh(**_MESH),
        scratch_types=_prop_scratch(F),
        compiler_params=_SC_PARAMS,
    )
    def dual(tables, gidx, sidx, zrows, out,
             gi0, gi1, si0, si1, rows0, rows1, acc, sem_g, ss):
        c = lax.axis_index("c")
        s = lax.axis_index("s")
        pltpu.sync_copy(zrows, acc.at[pl.ds(s * ZR, ZR)])
        plsc.subcore_barrier()
        base = s * rows_per_t

        def load_idx(j, gib, sib):
            r0 = base + j * KSUB
            pltpu.sync_copy(gidx.at[c, pl.ds(r0, KSUB)], gib)
            pltpu.sync_copy(sidx.at[c, pl.ds(r0, KSUB)], sib)

        bufs = (gi0, gi1, si0, si1, rows0, rows1, sem_g, ss)
        _edge_loop(load_idx, tables.at[c], acc, bufs, nchunks)
        plsc.subcore_barrier()
        pltpu.sync_copy(acc.at[pl.ds(s * ZR, ZR)], out.at[c, pl.ds(s * ZR, ZR)])

    return dual


@functools.lru_cache(maxsize=None)
def _make_deg(mpad):
    rows_per_t = mpad // SUB // NS
    nchunks = rows_per_t // KSUB

    @functools.partial(
        pl.kernel,
        out_type=jax.ShapeDtypeStruct((NC, NPAD, 16), jnp.float32),
        mesh=plsc.VectorSubcoreMesh(**_MESH),
        scratch_types=[
            pltpu.VMEM((KSUB, SUB), jnp.int32),
            pltpu.VMEM((KSUB, SUB), jnp.int32),
            pltpu.VMEM((SUB, 16), jnp.float32),
            pltpu.VMEM_SHARED((NPAD, 16), jnp.float32),
            pltpu.SemaphoreType.DMA,
        ],
        compiler_params=_SC_PARAMS,
    )
    def deg(dsts, ones_hbm, zrows, out, si0, si1, ones_v, acc, ss):
        c = lax.axis_index("c")
        s = lax.axis_index("s")
        pltpu.sync_copy(zrows, acc.at[pl.ds(s * ZR, ZR)])
        pltpu.sync_copy(ones_hbm, ones_v)
        plsc.subcore_barrier()
        base = s * rows_per_t

        def fire(j, sib):
            pltpu.sync_copy(dsts.at[c, pl.ds(base + j * KSUB, KSUB)], sib)
            for k in range(KSUB):
                pltpu.async_copy(ones_v, acc.at[sib.at[k]], ss, add=True)

        def drain(sib):
            for k in range(KSUB):
                pltpu.make_async_copy(ones_v, acc.at[sib.at[k]], ss).wait()

        def pair(t, carry):
            @pl.when(t >= 1)
            def _():
                drain(si0)

            fire(2 * t, si0)

            @pl.when(t >= 1)
            def _():
                drain(si1)

            fire(2 * t + 1, si1)
            return carry

        lax.fori_loop(0, nchunks // 2, pair, 0)
        drain(si0)
        drain(si1)
        plsc.subcore_barrier()
        pltpu.sync_copy(acc.at[pl.ds(s * ZR, ZR)], out.at[c, pl.ds(s * ZR, ZR)])

    return deg


BN = 1000
GRID = N // BN


def _row_spec(width):
    return pl.BlockSpec((BN, width), lambda i: (i, 0))


def _part_spec(width):
    return pl.BlockSpec((NC, BN, width), lambda i: (0, i, 0))


def _full_spec(shape):
    nd = len(shape)
    return pl.BlockSpec(shape, lambda i: (0,) * nd)


def _tc_call(body, in_specs, out_widths):
    return pl.pallas_call(
        body,
        grid=(GRID,),
        in_specs=in_specs,
        out_specs=tuple(_row_spec(w) for w in out_widths),
        out_shape=tuple(
            jax.ShapeDtypeStruct((N, w), jnp.float32) for w in out_widths
        ),
    )


def _tc_mm_body(x1, x2, w1, w2, h1o, h2o):
    h1o[...] = jnp.dot(x1[...], w1[...], preferred_element_type=jnp.float32)
    h2o[...] = jnp.dot(x2[...], w2[...], preferred_element_type=jnp.float32)


def _tc_scale_body(cnt, h1, h2, d1o, d2o, g1o, g2o):
    c = cnt[...]
    d1 = lax.rsqrt(c[0, :, 0:1] + 1.0)
    d2 = lax.rsqrt(c[1, :, 0:1] + 1.0)
    d1o[...] = d1
    d2o[...] = d2
    g1o[...] = d1 * h1[...]
    g2o[...] = d2 * h2[...]


def _tc_conv2_body(g2, pab, d2, b2, wog, g1, d1, b1, l2o, gogo, l1ao):
    l2 = jnp.maximum(d2[...] * (g2[...] + pab[0]) + b2[...], 0.0)
    l2o[...] = l2
    gogo[...] = d2[...] * jnp.dot(l2, wog[...], preferred_element_type=jnp.float32)
    l1ao[...] = jnp.maximum(d1[...] * (g1[...] + pab[1]) + b1[...], 0.0)


def _tc_mid_body(l1a, pcq, gog, d2, bog, l1o, f2o):
    l1o[...] = l1a[...] + pcq[1]
    f2o[...] = jnp.maximum(d2[...] * (gog[...] + pcq[0]) + bog[...], 0.0)


def _tc_emb_body(f2, r, wp1, d2, gp1o):
    xe = f2[...] + r[0] + r[1]
    gp1o[...] = d2[...] * jnp.dot(xe, wp1[...], preferred_element_type=jnp.float32)


def _tc_hid_body(gp1, pd, d2, bp1, wp2, gwo):
    h = jnp.maximum(d2[...] * (gp1[...] + pd[0] + pd[1]) + bp1[...], 0.0)
    gw = jnp.dot(d2[...] * h, wp2[...], preferred_element_type=jnp.float32)
    gwo[...] = jnp.concatenate([gw, jnp.zeros((BN, 14), jnp.float32)], axis=1)


def _tc_out_body(gw, pe, d2, bp2, outo):
    y = d2[...] * (gw[...] + pe[0] + pe[1])
    outo[...] = y[:, 0:2] + bp2[...]


def kernel(x_0, x_1, x_2, edge_index_0, edge_index_1, edge_index_2,
           layer_edge_index_0, layer_edge_index_1, layer_edge_index_2,
           W_lg_0, b_lg_0, W_lg_1, b_lg_1, W_lg_2, b_lg_2,
           W_og_0, b_og_0, W_og_1, b_og_1, W_og_2, b_og_2,
           W_p1, b_p1, W_p2, b_p2):
    ei1 = edge_index_1.astype(jnp.int32)
    ei2 = edge_index_2.astype(jnp.int32)
    lei2 = layer_edge_index_2.astype(jnp.int32)

    e_pad = _pad_len(ei2.shape[1])
    el_pad = _pad_len(lei2.shape[1])

    src1 = _pad_gather_idx(ei1[0], e_pad)
    dst1 = _pad_scatter_idx(ei1[1], e_pad)
    src2 = _pad_gather_idx(ei2[0], e_pad)
    dst2 = _pad_scatter_idx(ei2[1], e_pad)
    lg_in = _pad_gather_idx(lei2[0], el_pad)
    ls_in = _pad_scatter_idx(lei2[1], el_pad)
    lg_out_e = _pad_gather_idx(lei2[1], e_pad)
    ls_out_e = _pad_scatter_idx(lei2[0], e_pad)

    dsts = jnp.stack([dst1, dst2])

    zrows32 = jnp.zeros((ZR, F), jnp.float32)
    zrows16 = jnp.zeros((ZR, 16), jnp.float32)
    ones128 = jnp.ones((SUB, 16), jnp.float32)

    cnt = _make_deg(e_pad)(dsts, ones128, zrows16)

    h1, h2 = _tc_call(
        _tc_mm_body,
        [_row_spec(F), _row_spec(F), _full_spec((F, F)), _full_spec((F, F))],
        (F, F),
    )(x_1, x_2, W_lg_1, W_lg_2)

    d1, d2, g1, g2 = _tc_call(
        _tc_scale_body,
        [_part_spec(16), _row_spec(F), _row_spec(F)],
        (1, 1, F, F),
    )(cnt, h1, h2)

    dual = _make_dual(e_pad)
    pab = dual(jnp.stack([g2, g1]), jnp.stack([src2, src1]),
               jnp.stack([dst2, dst1]), zrows32)

    blg1 = b_lg_1.reshape(1, F)
    blg2 = b_lg_2.reshape(1, F)
    bog2 = b_og_2.reshape(1, F)
    bp1 = b_p1.reshape(1, F)

    l2, gog, l1a = _tc_call(
        _tc_conv2_body,
        [_row_spec(F), _part_spec(F), _row_spec(1), _full_spec((1, F)),
         _full_spec((F, F)), _row_spec(F), _row_spec(1), _full_spec((1, F))],
        (F, F, F),
    )(g2, pab, d2, blg2, W_og_2, g1, d1, blg1)

    pcq = dual(jnp.stack([gog, l2]), jnp.stack([src2, lg_out_e]),
               jnp.stack([dst2, ls_out_e]), zrows32)

    l1, f2 = _tc_call(
        _tc_mid_body,
        [_row_spec(F), _part_spec(F), _row_spec(F), _row_spec(1),
         _full_spec((1, F))],
        (F, F),
    )(l1a, pcq, gog, d2, bog2)

    r = _make_prop(el_pad, F)(l1, lg_in, ls_in, zrows32)

    gp1, = _tc_call(
        _tc_emb_body,
        [_row_spec(F), _part_spec(F), _full_spec((F, F)), _row_spec(1)],
        (F,),
    )(f2, r, W_p1, d2)

    pd = _make_prop(e_pad, F)(gp1, src2, dst2, zrows32)

    gw, = _tc_call(
        _tc_hid_body,
        [_row_spec(F), _part_spec(F), _row_spec(1), _full_spec((1, F)),
         _full_spec((F, 2))],
        (16,),
    )(gp1, pd, d2, bp1, W_p2)

    pe = _make_prop(e_pad, 16)(gw, src2, dst2, zrows16)

    out, = _tc_call(
        _tc_out_body,
        [_row_spec(16), _part_spec(16), _row_spec(1), _full_spec((1, 2))],
        (2,),
    )(gw, pe, d2, b_p2.reshape(1, 2))

    return out


# speedup vs baseline: 1.1219x; 1.1219x over previous
import functools

import jax
import jax.numpy as jnp
from jax import lax
from jax.experimental import pallas as pl
from jax.experimental.pallas import tpu as pltpu
from jax.experimental.pallas import tpu_sc as plsc

N = 50000
F = 32
NC = 2
NS = 16
NW = NC * NS
CH = 512
NPAD = 50048
ZR = NPAD // NS

_MESH = dict(core_axis_name="c", subcore_axis_name="s")
_SC_PARAMS = pltpu.CompilerParams(use_tc_tiling_on_sc=False)


def _pad_len(m):
    blk = NW * CH
    return ((m + blk - 1) // blk) * blk


def _pad_gather_idx(idx, mpad):
    pad = mpad - idx.shape[0]
    fill = lax.iota(jnp.int32, pad) % 128
    return jnp.concatenate([idx.astype(jnp.int32), fill]).reshape(mpad // CH, CH)


def _pad_scatter_idx(idx, mpad):
    pad = mpad - idx.shape[0]
    fill = N + (lax.iota(jnp.int32, pad) % 16)
    return jnp.concatenate([idx.astype(jnp.int32), fill]).reshape(mpad // CH, CH)


def _edge_loop(gidx, sidx, base, gather_tab, acc, gi, si, rows, sem_g, ss, nch):
    def chunk(j, carry):
        @pl.when(j >= 1)
        def _():
            pltpu.make_async_copy(rows, acc.at[si], ss).wait()

        pltpu.sync_copy(gidx(base + j), gi)
        pltpu.sync_copy(sidx(base + j), si)
        pltpu.async_copy(gather_tab.at[gi], rows, sem_g).wait()
        pltpu.async_copy(rows, acc.at[si], ss, add=True)
        return carry

    lax.fori_loop(0, nch, chunk, 0)
    pltpu.make_async_copy(rows, acc.at[si], ss).wait()


def _prop_scratch(width):
    return [
        pltpu.VMEM((CH,), jnp.int32),
        pltpu.VMEM((CH,), jnp.int32),
        pltpu.VMEM((CH, width), jnp.float32),
        pltpu.VMEM_SHARED((NPAD, width), jnp.float32),
        pltpu.SemaphoreType.DMA,
        pltpu.SemaphoreType.DMA,
    ]


@functools.lru_cache(maxsize=None)
def _make_prop(mpad, width):
    nch = mpad // CH // NW

    @functools.partial(
        pl.kernel,
        out_type=jax.ShapeDtypeStruct((NC, NPAD, width), jnp.float32),
        mesh=plsc.VectorSubcoreMesh(**_MESH),
        scratch_types=_prop_scratch(width),
        compiler_params=_SC_PARAMS,
    )
    def prop(table, gidx, sidx, zrows, out, gi, si, rows, acc, sem_g, ss):
        c = lax.axis_index("c")
        s = lax.axis_index("s")
        w = c * NS + s
        pltpu.sync_copy(zrows, acc.at[pl.ds(s * ZR, ZR)])
        plsc.subcore_barrier()
        _edge_loop(lambda r: gidx.at[r], lambda r: sidx.at[r], w * nch,
                   table, acc, gi, si, rows, sem_g, ss, nch)
        plsc.subcore_barrier()
        pltpu.sync_copy(acc.at[pl.ds(s * ZR, ZR)], out.at[c, pl.ds(s * ZR, ZR)])

    return prop


@functools.lru_cache(maxsize=None)
def _make_dual(mpad):
    nch = mpad // CH // NS

    @functools.partial(
        pl.kernel,
        out_type=jax.ShapeDtypeStruct((NC, NPAD, F), jnp.float32),
        mesh=plsc.VectorSubcoreMesh(**_MESH),
        scratch_types=_prop_scratch(F),
        compiler_params=_SC_PARAMS,
    )
    def dual(tables, gidx, sidx, zrows, out, gi, si, rows, acc, sem_g, ss):
        c = lax.axis_index("c")
        s = lax.axis_index("s")
        pltpu.sync_copy(zrows, acc.at[pl.ds(s * ZR, ZR)])
        plsc.subcore_barrier()
        _edge_loop(lambda r: gidx.at[c, r], lambda r: sidx.at[c, r], s * nch,
                   tables.at[c], acc, gi, si, rows, sem_g, ss, nch)
        plsc.subcore_barrier()
        pltpu.sync_copy(acc.at[pl.ds(s * ZR, ZR)], out.at[c, pl.ds(s * ZR, ZR)])

    return dual


@functools.lru_cache(maxsize=None)
def _make_deg(mpad):
    nch = mpad // CH // NS

    @functools.partial(
        pl.kernel,
        out_type=jax.ShapeDtypeStruct((NC, NPAD, 16), jnp.float32),
        mesh=plsc.VectorSubcoreMesh(**_MESH),
        scratch_types=[
            pltpu.VMEM((CH,), jnp.int32),
            pltpu.VMEM((CH, 16), jnp.float32),
            pltpu.VMEM_SHARED((NPAD, 16), jnp.float32),
            pltpu.SemaphoreType.DMA,
        ],
        compiler_params=_SC_PARAMS,
    )
    def deg(dsts, ones_hbm, zrows, out, si, ones_v, acc, ss):
        c = lax.axis_index("c")
        s = lax.axis_index("s")
        pltpu.sync_copy(zrows, acc.at[pl.ds(s * ZR, ZR)])
        pltpu.sync_copy(ones_hbm, ones_v)
        plsc.subcore_barrier()
        base = s * nch

        def chunk(j, carry):
            @pl.when(j >= 1)
            def _():
                pltpu.make_async_copy(ones_v, acc.at[si], ss).wait()

            pltpu.sync_copy(dsts.at[c, base + j], si)
            pltpu.async_copy(ones_v, acc.at[si], ss, add=True)
            return carry

        lax.fori_loop(0, nch, chunk, 0)
        pltpu.make_async_copy(ones_v, acc.at[si], ss).wait()
        plsc.subcore_barrier()
        pltpu.sync_copy(acc.at[pl.ds(s * ZR, ZR)], out.at[c, pl.ds(s * ZR, ZR)])

    return deg


BN = 1000
GRID = N // BN


def _row_spec(width):
    return pl.BlockSpec((BN, width), lambda i: (i, 0))


def _part_spec(width):
    return pl.BlockSpec((NC, BN, width), lambda i: (0, i, 0))


def _full_spec(shape):
    nd = len(shape)
    return pl.BlockSpec(shape, lambda i: (0,) * nd)


def _tc_call(body, in_specs, out_widths):
    return pl.pallas_call(
        body,
        grid=(GRID,),
        in_specs=in_specs,
        out_specs=tuple(_row_spec(w) for w in out_widths),
        out_shape=tuple(
            jax.ShapeDtypeStruct((N, w), jnp.float32) for w in out_widths
        ),
    )


def _tc_mm_body(x1, x2, w1, w2, h1o, h2o):
    h1o[...] = jnp.dot(x1[...], w1[...], preferred_element_type=jnp.float32)
    h2o[...] = jnp.dot(x2[...], w2[...], preferred_element_type=jnp.float32)


def _tc_scale_body(cnt, h1, h2, d1o, d2o, g1o, g2o):
    c = cnt[...]
    d1 = lax.rsqrt(c[0, :, 0:1] + 1.0)
    d2 = lax.rsqrt(c[1, :, 0:1] + 1.0)
    d1o[...] = d1
    d2o[...] = d2
    g1o[...] = d1 * h1[...]
    g2o[...] = d2 * h2[...]


def _tc_conv2_body(g2, pab, d2, b2, wog, g1, d1, b1, l2o, gogo, l1ao):
    l2 = jnp.maximum(d2[...] * (g2[...] + pab[0]) + b2[...], 0.0)
    l2o[...] = l2
    gogo[...] = d2[...] * jnp.dot(l2, wog[...], preferred_element_type=jnp.float32)
    l1ao[...] = jnp.maximum(d1[...] * (g1[...] + pab[1]) + b1[...], 0.0)


def _tc_mid_body(l1a, pcq, gog, d2, bog, l1o, f2o):
    l1o[...] = l1a[...] + pcq[1]
    f2o[...] = jnp.maximum(d2[...] * (gog[...] + pcq[0]) + bog[...], 0.0)


def _tc_emb_body(f2, r, wp1, d2, gp1o):
    xe = f2[...] + r[0] + r[1]
    gp1o[...] = d2[...] * jnp.dot(xe, wp1[...], preferred_element_type=jnp.float32)


def _tc_hid_body(gp1, pd, d2, bp1, wp2, gwo):
    h = jnp.maximum(d2[...] * (gp1[...] + pd[0] + pd[1]) + bp1[...], 0.0)
    gw = jnp.dot(d2[...] * h, wp2[...], preferred_element_type=jnp.float32)
    gwo[...] = jnp.concatenate([gw, jnp.zeros((BN, 14), jnp.float32)], axis=1)


def _tc_out_body(gw, pe, d2, bp2, outo):
    y = d2[...] * (gw[...] + pe[0] + pe[1])
    outo[...] = y[:, 0:2] + bp2[...]


def kernel(x_0, x_1, x_2, edge_index_0, edge_index_1, edge_index_2,
           layer_edge_index_0, layer_edge_index_1, layer_edge_index_2,
           W_lg_0, b_lg_0, W_lg_1, b_lg_1, W_lg_2, b_lg_2,
           W_og_0, b_og_0, W_og_1, b_og_1, W_og_2, b_og_2,
           W_p1, b_p1, W_p2, b_p2):
    ei1 = edge_index_1.astype(jnp.int32)
    ei2 = edge_index_2.astype(jnp.int32)
    lei2 = layer_edge_index_2.astype(jnp.int32)

    e_pad = _pad_len(ei2.shape[1])
    el_pad = _pad_len(lei2.shape[1])

    src1 = _pad_gather_idx(ei1[0], e_pad)
    dst1 = _pad_scatter_idx(ei1[1], e_pad)
    src2 = _pad_gather_idx(ei2[0], e_pad)
    dst2 = _pad_scatter_idx(ei2[1], e_pad)
    lg_in = _pad_gather_idx(lei2[0], el_pad)
    ls_in = _pad_scatter_idx(lei2[1], el_pad)
    lg_out_e = _pad_gather_idx(lei2[1], e_pad)
    ls_out_e = _pad_scatter_idx(lei2[0], e_pad)

    dsts = jnp.stack([dst1, dst2])

    zrows32 = jnp.zeros((ZR, F), jnp.float32)
    zrows16 = jnp.zeros((ZR, 16), jnp.float32)
    ones128 = jnp.ones((CH, 16), jnp.float32)

    cnt = _make_deg(e_pad)(dsts, ones128, zrows16)

    h1, h2 = _tc_call(
        _tc_mm_body,
        [_row_spec(F), _row_spec(F), _full_spec((F, F)), _full_spec((F, F))],
        (F, F),
    )(x_1, x_2, W_lg_1, W_lg_2)

    d1, d2, g1, g2 = _tc_call(
        _tc_scale_body,
        [_part_spec(16), _row_spec(F), _row_spec(F)],
        (1, 1, F, F),
    )(cnt, h1, h2)

    dual = _make_dual(e_pad)
    pab = dual(jnp.stack([g2, g1]), jnp.stack([src2, src1]),
               jnp.stack([dst2, dst1]), zrows32)

    blg1 = b_lg_1.reshape(1, F)
    blg2 = b_lg_2.reshape(1, F)
    bog2 = b_og_2.reshape(1, F)
    bp1 = b_p1.reshape(1, F)

    l2, gog, l1a = _tc_call(
        _tc_conv2_body,
        [_row_spec(F), _part_spec(F), _row_spec(1), _full_spec((1, F)),
         _full_spec((F, F)), _row_spec(F), _row_spec(1), _full_spec((1, F))],
        (F, F, F),
    )(g2, pab, d2, blg2, W_og_2, g1, d1, blg1)

    pcq = dual(jnp.stack([gog, l2]), jnp.stack([src2, lg_out_e]),
               jnp.stack([dst2, ls_out_e]), zrows32)

    l1, f2 = _tc_call(
        _tc_mid_body,
        [_row_spec(F), _part_spec(F), _row_spec(F), _row_spec(1),
         _full_spec((1, F))],
        (F, F),
    )(l1a, pcq, gog, d2, bog2)

    r = _make_prop(el_pad, F)(l1, lg_in, ls_in, zrows32)

    gp1, = _tc_call(
        _tc_emb_body,
        [_row_spec(F), _part_spec(F), _full_spec((F, F)), _row_spec(1)],
        (F,),
    )(f2, r, W_p1, d2)

    pd = _make_prop(e_pad, F)(gp1, src2, dst2, zrows32)

    gw, = _tc_call(
        _tc_hid_body,
        [_row_spec(F), _part_spec(F), _row_spec(1), _full_spec((1, F)),
         _full_spec((F, 2))],
        (16,),
    )(gp1, pd, d2, bp1, W_p2)

    pe = _make_prop(e_pad, 16)(gw, src2, dst2, zrows16)

    out, = _tc_call(
        _tc_out_body,
        [_row_spec(16), _part_spec(16), _row_spec(1), _full_spec((1, 2))],
        (2,),
    )(gw, pe, d2, b_p2.reshape(1, 2))

    return out


# speedup vs baseline: 1.2070x; 1.0759x over previous
import functools

import jax
import jax.numpy as jnp
from jax import lax
from jax.experimental import pallas as pl
from jax.experimental.pallas import tpu as pltpu
from jax.experimental.pallas import tpu_sc as plsc

N = 50000
F = 32
NC = 2
NS = 16
NW = NC * NS
CH = 512
NPAD = 50048
ZR = NPAD // NS

_MESH = dict(core_axis_name="c", subcore_axis_name="s")
_SC_PARAMS = pltpu.CompilerParams(use_tc_tiling_on_sc=False)


def _pad_len(m):
    blk = NW * CH
    return ((m + blk - 1) // blk) * blk


def _pad_gather_idx(idx, mpad):
    pad = mpad - idx.shape[0]
    fill = lax.iota(jnp.int32, pad) % 128
    return jnp.concatenate([idx.astype(jnp.int32), fill]).reshape(mpad // CH, CH)


def _pad_scatter_idx(idx, mpad):
    pad = mpad - idx.shape[0]
    fill = N + (lax.iota(jnp.int32, pad) % 16)
    return jnp.concatenate([idx.astype(jnp.int32), fill]).reshape(mpad // CH, CH)


def _edge_loop(gidx, sidx, base, gather_tab, acc, gi, si, rows, sem_g, ss, nch):
    def chunk(j, carry):
        @pl.when(j >= 1)
        def _():
            pltpu.make_async_copy(rows, acc.at[si], ss).wait()

        pltpu.sync_copy(gidx(base + j), gi)
        pltpu.sync_copy(sidx(base + j), si)
        pltpu.async_copy(gather_tab.at[gi], rows, sem_g).wait()
        pltpu.async_copy(rows, acc.at[si], ss, add=True)
        return carry

    lax.fori_loop(0, nch, chunk, 0)
    pltpu.make_async_copy(rows, acc.at[si], ss).wait()


def _prop_scratch(width):
    return [
        pltpu.VMEM((CH,), jnp.int32),
        pltpu.VMEM((CH,), jnp.int32),
        pltpu.VMEM((CH, width), jnp.float32),
        pltpu.VMEM_SHARED((NPAD, width), jnp.float32),
        pltpu.SemaphoreType.DMA,
        pltpu.SemaphoreType.DMA,
    ]


@functools.lru_cache(maxsize=None)
def _make_prop(mpad, width):
    nch = mpad // CH // NW

    @functools.partial(
        pl.kernel,
        out_type=jax.ShapeDtypeStruct((NC, NPAD, width), jnp.float32),
        mesh=plsc.VectorSubcoreMesh(**_MESH),
        scratch_types=_prop_scratch(width),
        compiler_params=_SC_PARAMS,
    )
    def prop(table, gidx, sidx, zrows, out, gi, si, rows, acc, sem_g, ss):
        c = lax.axis_index("c")
        s = lax.axis_index("s")
        w = c * NS + s
        pltpu.sync_copy(zrows, acc.at[pl.ds(s * ZR, ZR)])
        plsc.subcore_barrier()
        _edge_loop(lambda r: gidx.at[r], lambda r: sidx.at[r], w * nch,
                   table, acc, gi, si, rows, sem_g, ss, nch)
        plsc.subcore_barrier()
        pltpu.sync_copy(acc.at[pl.ds(s * ZR, ZR)], out.at[c, pl.ds(s * ZR, ZR)])

    return prop


@functools.lru_cache(maxsize=None)
def _make_dual(mpad):
    nch = mpad // CH // NS

    @functools.partial(
        pl.kernel,
        out_type=jax.ShapeDtypeStruct((NC, NPAD, F), jnp.float32),
        mesh=plsc.VectorSubcoreMesh(**_MESH),
        scratch_types=_prop_scratch(F),
        compiler_params=_SC_PARAMS,
    )
    def dual(tab_a, gidx_a, sidx_a, tab_b, gidx_b, sidx_b, zrows, out,
             gi, si, rows, acc, sem_g, ss):
        c = lax.axis_index("c")
        s = lax.axis_index("s")
        pltpu.sync_copy(zrows, acc.at[pl.ds(s * ZR, ZR)])
        plsc.subcore_barrier()
        base = s * nch

        def chunk(j, carry):
            @pl.when(j >= 1)
            def _():
                pltpu.make_async_copy(rows, acc.at[si], ss).wait()

            @pl.when(c == 0)
            def _():
                pltpu.sync_copy(gidx_a.at[base + j], gi)
                pltpu.sync_copy(sidx_a.at[base + j], si)
                pltpu.async_copy(tab_a.at[gi], rows, sem_g)

            @pl.when(c == 1)
            def _():
                pltpu.sync_copy(gidx_b.at[base + j], gi)
                pltpu.sync_copy(sidx_b.at[base + j], si)
                pltpu.async_copy(tab_b.at[gi], rows, sem_g)

            pltpu.make_async_copy(tab_a.at[gi], rows, sem_g).wait()
            pltpu.async_copy(rows, acc.at[si], ss, add=True)
            return carry

        lax.fori_loop(0, nch, chunk, 0)
        pltpu.make_async_copy(rows, acc.at[si], ss).wait()
        plsc.subcore_barrier()
        pltpu.sync_copy(acc.at[pl.ds(s * ZR, ZR)], out.at[c, pl.ds(s * ZR, ZR)])

    return dual


@functools.lru_cache(maxsize=None)
def _make_deg(mpad):
    nch = mpad // CH // NS

    @functools.partial(
        pl.kernel,
        out_type=jax.ShapeDtypeStruct((NC, NPAD, 16), jnp.float32),
        mesh=plsc.VectorSubcoreMesh(**_MESH),
        scratch_types=[
            pltpu.VMEM((CH,), jnp.int32),
            pltpu.VMEM((CH, 16), jnp.float32),
            pltpu.VMEM_SHARED((NPAD, 16), jnp.float32),
            pltpu.SemaphoreType.DMA,
        ],
        compiler_params=_SC_PARAMS,
    )
    def deg(dst_a, dst_b, ones_hbm, zrows, out, si, ones_v, acc, ss):
        c = lax.axis_index("c")
        s = lax.axis_index("s")
        pltpu.sync_copy(zrows, acc.at[pl.ds(s * ZR, ZR)])
        pltpu.sync_copy(ones_hbm, ones_v)
        plsc.subcore_barrier()
        base = s * nch

        def chunk(j, carry):
            @pl.when(j >= 1)
            def _():
                pltpu.make_async_copy(ones_v, acc.at[si], ss).wait()

            @pl.when(c == 0)
            def _():
                pltpu.sync_copy(dst_a.at[base + j], si)

            @pl.when(c == 1)
            def _():
                pltpu.sync_copy(dst_b.at[base + j], si)

            pltpu.async_copy(ones_v, acc.at[si], ss, add=True)
            return carry

        lax.fori_loop(0, nch, chunk, 0)
        pltpu.make_async_copy(ones_v, acc.at[si], ss).wait()
        plsc.subcore_barrier()
        pltpu.sync_copy(acc.at[pl.ds(s * ZR, ZR)], out.at[c, pl.ds(s * ZR, ZR)])

    return deg


BN = 1000
GRID = N // BN


def _row_spec(width):
    return pl.BlockSpec((BN, width), lambda i: (i, 0))


def _part_spec(width):
    return pl.BlockSpec((NC, BN, width), lambda i: (0, i, 0))


def _full_spec(shape):
    nd = len(shape)
    return pl.BlockSpec(shape, lambda i: (0,) * nd)


def _tc_call(body, in_specs, out_widths):
    return pl.pallas_call(
        body,
        grid=(GRID,),
        in_specs=in_specs,
        out_specs=tuple(_row_spec(w) for w in out_widths),
        out_shape=tuple(
            jax.ShapeDtypeStruct((N, w), jnp.float32) for w in out_widths
        ),
    )


def _tc_mm_body(x1, x2, w1, w2, h1o, h2o):
    h1o[...] = jnp.dot(x1[...], w1[...], preferred_element_type=jnp.float32)
    h2o[...] = jnp.dot(x2[...], w2[...], preferred_element_type=jnp.float32)


def _tc_scale_body(cnt, h1, h2, d1o, d2o, g1o, g2o):
    c = cnt[...]
    d1 = lax.rsqrt(c[0, :, 0:1] + 1.0)
    d2 = lax.rsqrt(c[1, :, 0:1] + 1.0)
    d1o[...] = d1
    d2o[...] = d2
    g1o[...] = d1 * h1[...]
    g2o[...] = d2 * h2[...]


def _tc_conv2_body(g2, pab, d2, b2, wog, g1, d1, b1, l2o, gogo, l1ao):
    l2 = jnp.maximum(d2[...] * (g2[...] + pab[0]) + b2[...], 0.0)
    l2o[...] = l2
    gogo[...] = d2[...] * jnp.dot(l2, wog[...], preferred_element_type=jnp.float32)
    l1ao[...] = jnp.maximum(d1[...] * (g1[...] + pab[1]) + b1[...], 0.0)


def _tc_mid_body(l1a, pcq, gog, d2, bog, l1o, f2o):
    l1o[...] = l1a[...] + pcq[1]
    f2o[...] = jnp.maximum(d2[...] * (gog[...] + pcq[0]) + bog[...], 0.0)


def _tc_emb_body(f2, r, wp1, d2, gp1o):
    xe = f2[...] + r[0] + r[1]
    gp1o[...] = d2[...] * jnp.dot(xe, wp1[...], preferred_element_type=jnp.float32)


def _tc_hid_body(gp1, pd, d2, bp1, wp2, gwo):
    h = jnp.maximum(d2[...] * (gp1[...] + pd[0] + pd[1]) + bp1[...], 0.0)
    gw = jnp.dot(d2[...] * h, wp2[...], preferred_element_type=jnp.float32)
    gwo[...] = jnp.concatenate([gw, jnp.zeros((BN, 14), jnp.float32)], axis=1)


def _tc_out_body(gw, pe, d2, bp2, outo):
    y = d2[...] * (gw[...] + pe[0] + pe[1])
    outo[...] = y[:, 0:2] + bp2[...]


def kernel(x_0, x_1, x_2, edge_index_0, edge_index_1, edge_index_2,
           layer_edge_index_0, layer_edge_index_1, layer_edge_index_2,
           W_lg_0, b_lg_0, W_lg_1, b_lg_1, W_lg_2, b_lg_2,
           W_og_0, b_og_0, W_og_1, b_og_1, W_og_2, b_og_2,
           W_p1, b_p1, W_p2, b_p2):
    ei1 = edge_index_1.astype(jnp.int32)
    ei2 = edge_index_2.astype(jnp.int32)
    lei2 = layer_edge_index_2.astype(jnp.int32)

    e_pad = _pad_len(ei2.shape[1])
    el_pad = _pad_len(lei2.shape[1])

    src1 = _pad_gather_idx(ei1[0], e_pad)
    dst1 = _pad_scatter_idx(ei1[1], e_pad)
    src2 = _pad_gather_idx(ei2[0], e_pad)
    dst2 = _pad_scatter_idx(ei2[1], e_pad)
    lg_in = _pad_gather_idx(lei2[0], el_pad)
    ls_in = _pad_scatter_idx(lei2[1], el_pad)
    lg_out_e = _pad_gather_idx(lei2[1], e_pad)
    ls_out_e = _pad_scatter_idx(lei2[0], e_pad)

    zrows32 = jnp.zeros((ZR, F), jnp.float32)
    zrows16 = jnp.zeros((ZR, 16), jnp.float32)
    ones128 = jnp.ones((CH, 16), jnp.float32)

    cnt = _make_deg(e_pad)(dst1, dst2, ones128, zrows16)

    h1, h2 = _tc_call(
        _tc_mm_body,
        [_row_spec(F), _row_spec(F), _full_spec((F, F)), _full_spec((F, F))],
        (F, F),
    )(x_1, x_2, W_lg_1, W_lg_2)

    d1, d2, g1, g2 = _tc_call(
        _tc_scale_body,
        [_part_spec(16), _row_spec(F), _row_spec(F)],
        (1, 1, F, F),
    )(cnt, h1, h2)

    dual = _make_dual(e_pad)
    pab = dual(g2, src2, dst2, g1, src1, dst1, zrows32)

    blg1 = b_lg_1.reshape(1, F)
    blg2 = b_lg_2.reshape(1, F)
    bog2 = b_og_2.reshape(1, F)
    bp1 = b_p1.reshape(1, F)

    l2, gog, l1a = _tc_call(
        _tc_conv2_body,
        [_row_spec(F), _part_spec(F), _row_spec(1), _full_spec((1, F)),
         _full_spec((F, F)), _row_spec(F), _row_spec(1), _full_spec((1, F))],
        (F, F, F),
    )(g2, pab, d2, blg2, W_og_2, g1, d1, blg1)

    pcq = dual(gog, src2, dst2, l2, lg_out_e, ls_out_e, zrows32)

    l1, f2 = _tc_call(
        _tc_mid_body,
        [_row_spec(F), _part_spec(F), _row_spec(F), _row_spec(1),
         _full_spec((1, F))],
        (F, F),
    )(l1a, pcq, gog, d2, bog2)

    r = _make_prop(el_pad, F)(l1, lg_in, ls_in, zrows32)

    gp1, = _tc_call(
        _tc_emb_body,
        [_row_spec(F), _part_spec(F), _full_spec((F, F)), _row_spec(1)],
        (F,),
    )(f2, r, W_p1, d2)

    pd = _make_prop(e_pad, F)(gp1, src2, dst2, zrows32)

    gw, = _tc_call(
        _tc_hid_body,
        [_row_spec(F), _part_spec(F), _row_spec(1), _full_spec((1, F)),
         _full_spec((F, 2))],
        (16,),
    )(gp1, pd, d2, bp1, W_p2)

    pe = _make_prop(e_pad, 16)(gw, src2, dst2, zrows16)

    out, = _tc_call(
        _tc_out_body,
        [_row_spec(16), _part_spec(16), _row_spec(1), _full_spec((1, 2))],
        (2,),
    )(gw, pe, d2, b_p2.reshape(1, 2))

    return out
